# baseline (speedup 1.0000x reference)
import functools

import jax
import jax.numpy as jnp
from jax import lax
from jax.experimental import pallas as pl
from jax.experimental.pallas import tpu as pltpu
import jax.experimental.pallas.tpu_sc as plsc

N = 16384
K = 16
NHID = 128
NCLS = 40
B = 16
NF = 3

RB = 128
CB = 256
G = N // RB
NCB = N // CB
MMR = 1024
MG = N // MMR
IBIG = 2 ** 30
HIGH = jax.lax.Precision.HIGHEST


def _knn_kernel(lo_ref, hi_ref, xtra_ref, pos_ref, post_ref, brow_ref,
                bcol_ref, out_ref, rv_ref, ri_ref):
    i = pl.program_id(0)
    qx = post_ref[0:1, :]
    qy = post_ref[1:2, :]
    qz = post_ref[2:3, :]
    bq = bcol_ref[...]

    rv_ref[...] = jnp.full((K, RB), jnp.inf, jnp.float32)
    ri_ref[...] = jnp.full((K, RB), IBIG, jnp.int32)

    def merge_block(cb):
        r0 = pl.multiple_of(cb * CB, CB)
        p = pos_ref[pl.ds(r0, CB), :]
        bs = brow_ref[pl.ds(r0, CB), :]
        tx = p[:, 0:1] - qx
        ty = p[:, 1:2] - qy
        tz = p[:, 2:3] - qz
        d = (tx * tx + ty * ty) + tz * tz
        d = jnp.where(bs == bq, d, jnp.inf)
        sidx = lax.broadcasted_iota(jnp.int32, (CB, RB), 0)
        rv = rv_ref[...]
        ri = ri_ref[...]
        ov = []
        oi = []
        for _ in range(K):
            m = jnp.minimum(jnp.min(d, axis=0, keepdims=True),
                            jnp.min(rv, axis=0, keepdims=True))
            cid = jnp.minimum(
                r0 + jnp.min(jnp.where(d == m, sidx, IBIG), axis=0,
                             keepdims=True),
                jnp.min(jnp.where(rv == m, ri, IBIG), axis=0, keepdims=True))
            ov.append(m)
            oi.append(cid)
            d = jnp.where(sidx == cid - r0, jnp.inf, d)
            rv = jnp.where(ri == cid, jnp.inf, rv)
        rv_ref[...] = jnp.concatenate(ov, axis=0)
        ri_ref[...] = jnp.concatenate(oi, axis=0)

    lo = lo_ref[i]
    hi = hi_ref[i]
    s0 = lo // CB
    s1 = (hi - 1) // CB

    def body(cb, carry):
        merge_block(cb)
        return carry

    lax.fori_loop(s0, s1 + 1, body, 0)

    @pl.when(xtra_ref[i] > 0)
    def _():
        merge_block(0)

    out_ref[...] = ri_ref[...]


def _knn(lo, hi, xtra, pos, post, brow, bcol):
    return pl.pallas_call(
        _knn_kernel,
        grid=(G,),
        in_specs=[
            pl.BlockSpec(memory_space=pltpu.SMEM),
            pl.BlockSpec(memory_space=pltpu.SMEM),
            pl.BlockSpec(memory_space=pltpu.SMEM),
            pl.BlockSpec((N, NF), lambda i: (0, 0)),
            pl.BlockSpec((NF, RB), lambda i: (0, i)),
            pl.BlockSpec((N, 1), lambda i: (0, 0)),
            pl.BlockSpec((1, RB), lambda i: (0, i)),
        ],
        out_specs=pl.BlockSpec((K, RB), lambda i: (0, i)),
        out_shape=jax.ShapeDtypeStruct((K, N), jnp.int32),
        scratch_shapes=[
            pltpu.VMEM((K, RB), jnp.float32),
            pltpu.VMEM((K, RB), jnp.int32),
        ],
    )(lo, hi, xtra, pos, post, brow, bcol)


def _mm1_kernel(pos_ref, ma_ref, mb_ref, a_ref, b_ref):
    q = pos_ref[...]

    def lin(m_ref):
        m = m_ref[...]
        return (q[:, 0:1] * m[0:1, :] + q[:, 1:2] * m[1:2, :]) + \
            q[:, 2:3] * m[2:3, :]

    a_ref[...] = lin(ma_ref)
    b_ref[...] = lin(mb_ref)


def _mm1(pos, m1a, m1b):
    return pl.pallas_call(
        _mm1_kernel,
        grid=(MG,),
        in_specs=[
            pl.BlockSpec((MMR, NF), lambda i: (i, 0)),
            pl.BlockSpec((NF, NHID), lambda i: (0, 0)),
            pl.BlockSpec((NF, NHID), lambda i: (0, 0)),
        ],
        out_specs=[
            pl.BlockSpec((MMR, NHID), lambda i: (i, 0)),
            pl.BlockSpec((MMR, NHID), lambda i: (i, 0)),
        ],
        out_shape=[
            jax.ShapeDtypeStruct((N, NHID), jnp.float32),
            jax.ShapeDtypeStruct((N, NHID), jnp.float32),
        ],
    )(pos, m1a, m1b)


def _mm2_kernel(a_ref, g_ref, b1_ref, ma_ref, mb_ref, a2_ref, b2_ref):
    h = jnp.maximum(a_ref[...] + g_ref[...] + b1_ref[...], 0.0)
    a2_ref[...] = jnp.dot(h, ma_ref[...],
                          preferred_element_type=jnp.float32, precision=HIGH)
    b2_ref[...] = jnp.dot(h, mb_ref[...],
                          preferred_element_type=jnp.float32, precision=HIGH)


def _mm2(a1, g1, b1, m2a, m2b):
    return pl.pallas_call(
        _mm2_kernel,
        grid=(MG,),
        in_specs=[
            pl.BlockSpec((MMR, NHID), lambda i: (i, 0)),
            pl.BlockSpec((MMR, NHID), lambda i: (i, 0)),
            pl.BlockSpec((1, NHID), lambda i: (0, 0)),
            pl.BlockSpec((NHID, NHID), lambda i: (0, 0)),
            pl.BlockSpec((NHID, NHID), lambda i: (0, 0)),
        ],
        out_specs=[
            pl.BlockSpec((MMR, NHID), lambda i: (i, 0)),
            pl.BlockSpec((MMR, NHID), lambda i: (i, 0)),
        ],
        out_shape=[
            jax.ShapeDtypeStruct((N, NHID), jnp.float32),
            jax.ShapeDtypeStruct((N, NHID), jnp.float32),
        ],
    )(a1, g1, b1, m2a, m2b)


def _final_kernel(first_ref, last_ref, a_ref, g_ref, brow_ref, b2_ref,
                  wc_ref, bc_ref, out_ref, pool_ref):
    i = pl.program_id(0)

    @pl.when(i == 0)
    def _():
        pool_ref[...] = jnp.full((B, NHID), -jnp.inf, jnp.float32)

    h = jnp.maximum(a_ref[...] + g_ref[...] + b2_ref[...], 0.0)
    br = brow_ref[...]

    def body(bb, carry):
        mx = jnp.max(jnp.where(br == bb, h, -jnp.inf), axis=0, keepdims=True)
        pool_ref[pl.ds(bb, 1), :] = jnp.maximum(pool_ref[pl.ds(bb, 1), :], mx)
        return carry

    lax.fori_loop(first_ref[i], last_ref[i] + 1, body, 0)

    @pl.when(i == pl.num_programs(0) - 1)
    def _():
        out_ref[...] = jnp.dot(pool_ref[...], wc_ref[...],
                               preferred_element_type=jnp.float32,
                               precision=HIGH) + bc_ref[...]


def _final(first, last, a2, g2, brow, b2, wct, bc):
    return pl.pallas_call(
        _final_kernel,
        grid=(MG,),
        in_specs=[
            pl.BlockSpec(memory_space=pltpu.SMEM),
            pl.BlockSpec(memory_space=pltpu.SMEM),
            pl.BlockSpec((MMR, NHID), lambda i: (i, 0)),
            pl.BlockSpec((MMR, NHID), lambda i: (i, 0)),
            pl.BlockSpec((MMR, 1), lambda i: (i, 0)),
            pl.BlockSpec((1, NHID), lambda i: (0, 0)),
            pl.BlockSpec((NHID, NCLS), lambda i: (0, 0)),
            pl.BlockSpec((1, NCLS), lambda i: (0, 0)),
        ],
        out_specs=pl.BlockSpec((B, NCLS), lambda i: (0, 0)),
        out_shape=jax.ShapeDtypeStruct((B, NCLS), jnp.float32),
        scratch_shapes=[pltpu.VMEM((B, NHID), jnp.float32)],
    )(first, last, a2, g2, brow, b2, wct, bc)


_NC = 2
_NS = 16
_NW = _NC * _NS
_RPW = N // _NW
_CH = 8
_NCHUNK = _RPW // _CH

_NBUF = 4


def _scmax_body(nbr_ref, bm_ref, out_ref, idx_v, rows0, rows1, rows2, rows3,
                ob0, ob1, ob2, ob3, gsem0, gsem1, gsem2, gsem3,
                osem0, osem1, osem2, osem3):
    wid = lax.axis_index("s") * _NC + lax.axis_index("c")
    base = wid * _RPW
    rows = (rows0, rows1, rows2, rows3)
    obs = (ob0, ob1, ob2, ob3)
    gsems = (gsem0, gsem1, gsem2, gsem3)
    osems = (osem0, osem1, osem2, osem3)

    pltpu.sync_copy(nbr_ref.at[pl.ds(base * K, _RPW * K)], idx_v)

    def gstart(ch, buf, sem):
        pltpu.async_copy(bm_ref.at[idx_v.at[pl.ds(ch * _CH * K, _CH * K)]],
                         buf, sem)

    def gwait(buf, sem):
        pltpu.make_async_copy(bm_ref.at[idx_v.at[pl.ds(0, _CH * K)]],
                              buf, sem).wait()

    def owait(ob, sem):
        pltpu.make_async_copy(ob, out_ref.at[pl.ds(base, _CH)], sem).wait()

    def compute(rows_v, out_v):
        for dl in range(_CH):
            for g in range(NHID // 16):
                acc = rows_v[dl * K, pl.ds(g * 16, 16)]
                for j in range(1, K):
                    acc = jnp.maximum(acc,
                                      rows_v[dl * K + j, pl.ds(g * 16, 16)])
                out_v[dl, pl.ds(g * 16, 16)] = acc

    for b in range(_NBUF):
        gstart(b, rows[b], gsems[b])

    def grp(p, carry):
        ch0 = _NBUF * p
        for b in range(_NBUF):
            ch = ch0 + b
            gwait(rows[b], gsems[b])

            @pl.when(p > 0)
            def _():
                owait(obs[b], osems[b])

            compute(rows[b], obs[b])
            pltpu.async_copy(obs[b], out_ref.at[pl.ds(base + ch * _CH, _CH)],
                             osems[b])

            @pl.when(p < _NCHUNK // _NBUF - 1)
            def _():
                gstart(ch + _NBUF, rows[b], gsems[b])
        return carry

    lax.fori_loop(0, _NCHUNK // _NBUF, grp, 0)
    for b in range(_NBUF):
        owait(obs[b], osems[b])


@functools.lru_cache(maxsize=1)
def _scmax_fn():
    mesh = plsc.VectorSubcoreMesh(
        core_axis_name="c", subcore_axis_name="s",
        num_cores=_NC, num_subcores=_NS)
    return pl.kernel(
        _scmax_body,
        out_type=jax.ShapeDtypeStruct((N, NHID), jnp.float32),
        mesh=mesh,
        scratch_types=(
            [pltpu.VMEM((_RPW * K,), jnp.int32)]
            + [pltpu.VMEM((_CH * K, NHID), jnp.float32)] * _NBUF
            + [pltpu.VMEM((_CH, NHID), jnp.float32)] * _NBUF
            + [pltpu.SemaphoreType.DMA] * (2 * _NBUF)
        ),
    )


def _scmax(nbr_flat, bm):
    return _scmax_fn()(nbr_flat, bm)


def kernel(pos, batch, W1, b1, W2, b2, Wc, bc):
    pos = pos.astype(jnp.float32)
    batch = batch.astype(jnp.int32)

    post = pos.T
    brow = batch.reshape(N, 1)
    bcol = batch.reshape(1, N)

    bnd = jnp.searchsorted(
        batch, jnp.arange(B + 1, dtype=jnp.int32), side="left"
    ).astype(jnp.int32)
    ridx = jnp.arange(G, dtype=jnp.int32) * RB
    first = batch[ridx]
    last = batch[ridx + RB - 1]
    lo = bnd[first]
    hi = bnd[last + 1]
    seg = bnd[1:] - bnd[:-1]
    bb = jnp.arange(B, dtype=jnp.int32)
    inblk = (bb[None, :] >= first[:, None]) & (bb[None, :] <= last[:, None])
    minseg = jnp.min(jnp.where(inblk, seg[None, :], IBIG), axis=1)
    xtra = (minseg < K).astype(jnp.int32)

    m1a = (W1[:, :NF] - W1[:, NF:]).T
    m1b = W1[:, NF:].T
    m2a = (W2[:, :NHID] - W2[:, NHID:]).T
    m2b = W2[:, NHID:].T
    wct = Wc.T
    b1r = b1.reshape(1, NHID)
    b2r = b2.reshape(1, NHID)
    bcr = bc.reshape(1, NCLS)

    nbr = _knn(lo, hi, xtra, pos, post, brow, bcol)
    nbr_flat = nbr.T.reshape(-1)

    a1, bm1 = _mm1(pos, m1a, m1b)
    g1 = _scmax(nbr_flat, bm1)
    a2, bm2 = _mm2(a1, g1, b1r, m2a, m2b)
    g2 = _scmax(nbr_flat, bm2)

    midx = jnp.arange(MG, dtype=jnp.int32) * MMR
    f2 = batch[midx]
    l2 = batch[midx + MMR - 1]
    return _final(f2, l2, a2, g2, brow, b2r, wct, bcr)


# speedup vs baseline: 1.0738x; 1.0738x over previous
import functools

import jax
import jax.numpy as jnp
from jax import lax
from jax.experimental import pallas as pl
from jax.experimental.pallas import tpu as pltpu
import jax.experimental.pallas.tpu_sc as plsc

N = 16384
K = 16
NHID = 128
NCLS = 40
B = 16
NF = 3

RB = 128
CB = 512
G = N // RB
NCB = N // CB
MMR = 1024
MG = N // MMR
IBIG = 2 ** 30
HIGH = jax.lax.Precision.HIGHEST


def _knn_kernel(lo_ref, hi_ref, xtra_ref, pos_ref, post_ref, brow_ref,
                bcol_ref, out_ref, rv_ref, ri_ref):
    i = pl.program_id(0)
    qx = post_ref[0:1, :]
    qy = post_ref[1:2, :]
    qz = post_ref[2:3, :]
    bq = bcol_ref[...]

    rv_ref[...] = jnp.full((K, RB), jnp.inf, jnp.float32)
    ri_ref[...] = jnp.full((K, RB), IBIG, jnp.int32)

    def merge_block(cb):
        r0 = pl.multiple_of(cb * CB, CB)
        p = pos_ref[pl.ds(r0, CB), :]
        bs = brow_ref[pl.ds(r0, CB), :]
        tx = p[:, 0:1] - qx
        ty = p[:, 1:2] - qy
        tz = p[:, 2:3] - qz
        d = (tx * tx + ty * ty) + tz * tz
        d = jnp.where(bs == bq, d, jnp.inf)
        sidx = lax.broadcasted_iota(jnp.int32, (CB, RB), 0)
        rv = rv_ref[...]
        ri = ri_ref[...]
        ov = []
        oi = []
        for _ in range(K):
            m = jnp.minimum(jnp.min(d, axis=0, keepdims=True),
                            jnp.min(rv, axis=0, keepdims=True))
            cid = jnp.minimum(
                r0 + jnp.min(jnp.where(d == m, sidx, IBIG), axis=0,
                             keepdims=True),
                jnp.min(jnp.where(rv == m, ri, IBIG), axis=0, keepdims=True))
            ov.append(m)
            oi.append(cid)
            d = jnp.where(sidx == cid - r0, jnp.inf, d)
            rv = jnp.where(ri == cid, jnp.inf, rv)
        rv_ref[...] = jnp.concatenate(ov, axis=0)
        ri_ref[...] = jnp.concatenate(oi, axis=0)

    lo = lo_ref[i]
    hi = hi_ref[i]
    s0 = lo // CB
    s1 = (hi - 1) // CB

    def body(cb, carry):
        merge_block(cb)
        return carry

    lax.fori_loop(s0, s1 + 1, body, 0)

    @pl.when(xtra_ref[i] > 0)
    def _():
        merge_block(0)

    out_ref[...] = ri_ref[...]


def _knn(lo, hi, xtra, pos, post, brow, bcol):
    return pl.pallas_call(
        _knn_kernel,
        grid=(G,),
        in_specs=[
            pl.BlockSpec(memory_space=pltpu.SMEM),
            pl.BlockSpec(memory_space=pltpu.SMEM),
            pl.BlockSpec(memory_space=pltpu.SMEM),
            pl.BlockSpec((N, NF), lambda i: (0, 0)),
            pl.BlockSpec((NF, RB), lambda i: (0, i)),
            pl.BlockSpec((N, 1), lambda i: (0, 0)),
            pl.BlockSpec((1, RB), lambda i: (0, i)),
        ],
        out_specs=pl.BlockSpec((K, RB), lambda i: (0, i)),
        out_shape=jax.ShapeDtypeStruct((K, N), jnp.int32),
        scratch_shapes=[
            pltpu.VMEM((K, RB), jnp.float32),
            pltpu.VMEM((K, RB), jnp.int32),
        ],
    )(lo, hi, xtra, pos, post, brow, bcol)


def _mm1_kernel(pos_ref, ma_ref, mb_ref, a_ref, b_ref):
    q = pos_ref[...]

    def lin(m_ref):
        m = m_ref[...]
        return (q[:, 0:1] * m[0:1, :] + q[:, 1:2] * m[1:2, :]) + \
            q[:, 2:3] * m[2:3, :]

    a_ref[...] = lin(ma_ref)
    b_ref[...] = lin(mb_ref)


def _mm1(pos, m1a, m1b):
    return pl.pallas_call(
        _mm1_kernel,
        grid=(MG,),
        in_specs=[
            pl.BlockSpec((MMR, NF), lambda i: (i, 0)),
            pl.BlockSpec((NF, NHID), lambda i: (0, 0)),
            pl.BlockSpec((NF, NHID), lambda i: (0, 0)),
        ],
        out_specs=[
            pl.BlockSpec((MMR, NHID), lambda i: (i, 0)),
            pl.BlockSpec((MMR, NHID), lambda i: (i, 0)),
        ],
        out_shape=[
            jax.ShapeDtypeStruct((N, NHID), jnp.float32),
            jax.ShapeDtypeStruct((N, NHID), jnp.float32),
        ],
    )(pos, m1a, m1b)


def _mm2_kernel(a_ref, g_ref, b1_ref, ma_ref, mb_ref, a2_ref, b2_ref):
    h = jnp.maximum(a_ref[...] + g_ref[...] + b1_ref[...], 0.0)
    a2_ref[...] = jnp.dot(h, ma_ref[...],
                          preferred_element_type=jnp.float32, precision=HIGH)
    b2_ref[...] = jnp.dot(h, mb_ref[...],
                          preferred_element_type=jnp.float32, precision=HIGH)


def _mm2(a1, g1, b1, m2a, m2b):
    return pl.pallas_call(
        _mm2_kernel,
        grid=(MG,),
        in_specs=[
            pl.BlockSpec((MMR, NHID), lambda i: (i, 0)),
            pl.BlockSpec((MMR, NHID), lambda i: (i, 0)),
            pl.BlockSpec((1, NHID), lambda i: (0, 0)),
            pl.BlockSpec((NHID, NHID), lambda i: (0, 0)),
            pl.BlockSpec((NHID, NHID), lambda i: (0, 0)),
        ],
        out_specs=[
            pl.BlockSpec((MMR, NHID), lambda i: (i, 0)),
            pl.BlockSpec((MMR, NHID), lambda i: (i, 0)),
        ],
        out_shape=[
            jax.ShapeDtypeStruct((N, NHID), jnp.float32),
            jax.ShapeDtypeStruct((N, NHID), jnp.float32),
        ],
    )(a1, g1, b1, m2a, m2b)


def _final_kernel(first_ref, last_ref, a_ref, g_ref, brow_ref, b2_ref,
                  wc_ref, bc_ref, out_ref, pool_ref):
    i = pl.program_id(0)

    @pl.when(i == 0)
    def _():
        pool_ref[...] = jnp.full((B, NHID), -jnp.inf, jnp.float32)

    h = jnp.maximum(a_ref[...] + g_ref[...] + b2_ref[...], 0.0)
    br = brow_ref[...]

    def body(bb, carry):
        mx = jnp.max(jnp.where(br == bb, h, -jnp.inf), axis=0, keepdims=True)
        pool_ref[pl.ds(bb, 1), :] = jnp.maximum(pool_ref[pl.ds(bb, 1), :], mx)
        return carry

    lax.fori_loop(first_ref[i], last_ref[i] + 1, body, 0)

    @pl.when(i == pl.num_programs(0) - 1)
    def _():
        out_ref[...] = jnp.dot(pool_ref[...], wc_ref[...],
                               preferred_element_type=jnp.float32,
                               precision=HIGH) + bc_ref[...]


def _final(first, last, a2, g2, brow, b2, wct, bc):
    return pl.pallas_call(
        _final_kernel,
        grid=(MG,),
        in_specs=[
            pl.BlockSpec(memory_space=pltpu.SMEM),
            pl.BlockSpec(memory_space=pltpu.SMEM),
            pl.BlockSpec((MMR, NHID), lambda i: (i, 0)),
            pl.BlockSpec((MMR, NHID), lambda i: (i, 0)),
            pl.BlockSpec((MMR, 1), lambda i: (i, 0)),
            pl.BlockSpec((1, NHID), lambda i: (0, 0)),
            pl.BlockSpec((NHID, NCLS), lambda i: (0, 0)),
            pl.BlockSpec((1, NCLS), lambda i: (0, 0)),
        ],
        out_specs=pl.BlockSpec((B, NCLS), lambda i: (0, 0)),
        out_shape=jax.ShapeDtypeStruct((B, NCLS), jnp.float32),
        scratch_shapes=[pltpu.VMEM((B, NHID), jnp.float32)],
    )(first, last, a2, g2, brow, b2, wct, bc)


_NC = 2
_NS = 16
_NW = _NC * _NS
_RPW = N // _NW
_CH = 8
_NCHUNK = _RPW // _CH

_NBUF = 2


def _scmax_body(nbr_ref, bm_ref, out_ref, idx_v, rows0, rows1,
                ob0, ob1, gsem0, gsem1, osem0, osem1):
    wid = lax.axis_index("s") * _NC + lax.axis_index("c")
    base = wid * _RPW
    rows = (rows0, rows1)
    obs = (ob0, ob1)
    gsems = (gsem0, gsem1)
    osems = (osem0, osem1)

    pltpu.sync_copy(nbr_ref.at[pl.ds(base * K, _RPW * K)], idx_v)

    def gstart(ch, buf, sem):
        pltpu.async_copy(bm_ref.at[idx_v.at[pl.ds(ch * _CH * K, _CH * K)]],
                         buf, sem)

    def gwait(buf, sem):
        pltpu.make_async_copy(bm_ref.at[idx_v.at[pl.ds(0, _CH * K)]],
                              buf, sem).wait()

    def owait(ob, sem):
        pltpu.make_async_copy(ob, out_ref.at[pl.ds(base, _CH)], sem).wait()

    def compute(rows_v, out_v):
        for dl in range(_CH):
            for g in range(NHID // 16):
                acc = rows_v[dl * K, pl.ds(g * 16, 16)]
                for j in range(1, K):
                    acc = jnp.maximum(acc,
                                      rows_v[dl * K + j, pl.ds(g * 16, 16)])
                out_v[dl, pl.ds(g * 16, 16)] = acc

    for b in range(_NBUF):
        gstart(b, rows[b], gsems[b])

    def grp(p, carry):
        ch0 = _NBUF * p
        for b in range(_NBUF):
            ch = ch0 + b
            gwait(rows[b], gsems[b])

            @pl.when(p > 0)
            def _():
                owait(obs[b], osems[b])

            compute(rows[b], obs[b])
            pltpu.async_copy(obs[b], out_ref.at[pl.ds(base + ch * _CH, _CH)],
                             osems[b])

            @pl.when(p < _NCHUNK // _NBUF - 1)
            def _():
                gstart(ch + _NBUF, rows[b], gsems[b])
        return carry

    lax.fori_loop(0, _NCHUNK // _NBUF, grp, 0)
    for b in range(_NBUF):
        owait(obs[b], osems[b])


@functools.lru_cache(maxsize=1)
def _scmax_fn():
    mesh = plsc.VectorSubcoreMesh(
        core_axis_name="c", subcore_axis_name="s",
        num_cores=_NC, num_subcores=_NS)
    return pl.kernel(
        _scmax_body,
        out_type=jax.ShapeDtypeStruct((N, NHID), jnp.float32),
        mesh=mesh,
        scratch_types=(
            [pltpu.VMEM((_RPW * K,), jnp.int32)]
            + [pltpu.VMEM((_CH * K, NHID), jnp.float32)] * _NBUF
            + [pltpu.VMEM((_CH, NHID), jnp.float32)] * _NBUF
            + [pltpu.SemaphoreType.DMA] * (2 * _NBUF)
        ),
    )


def _scmax(nbr_flat, bm):
    return _scmax_fn()(nbr_flat, bm)


def kernel(pos, batch, W1, b1, W2, b2, Wc, bc):
    pos = pos.astype(jnp.float32)
    batch = batch.astype(jnp.int32)

    post = pos.T
    brow = batch.reshape(N, 1)
    bcol = batch.reshape(1, N)

    bnd = jnp.searchsorted(
        batch, jnp.arange(B + 1, dtype=jnp.int32), side="left"
    ).astype(jnp.int32)
    ridx = jnp.arange(G, dtype=jnp.int32) * RB
    first = batch[ridx]
    last = batch[ridx + RB - 1]
    lo = bnd[first]
    hi = bnd[last + 1]
    seg = bnd[1:] - bnd[:-1]
    bb = jnp.arange(B, dtype=jnp.int32)
    inblk = (bb[None, :] >= first[:, None]) & (bb[None, :] <= last[:, None])
    minseg = jnp.min(jnp.where(inblk, seg[None, :], IBIG), axis=1)
    xtra = (minseg < K).astype(jnp.int32)

    m1a = (W1[:, :NF] - W1[:, NF:]).T
    m1b = W1[:, NF:].T
    m2a = (W2[:, :NHID] - W2[:, NHID:]).T
    m2b = W2[:, NHID:].T
    wct = Wc.T
    b1r = b1.reshape(1, NHID)
    b2r = b2.reshape(1, NHID)
    bcr = bc.reshape(1, NCLS)

    nbr = _knn(lo, hi, xtra, pos, post, brow, bcol)
    nbr_flat = nbr.T.reshape(-1)

    a1, bm1 = _mm1(pos, m1a, m1b)
    g1 = _scmax(nbr_flat, bm1)
    a2, bm2 = _mm2(a1, g1, b1r, m2a, m2b)
    g2 = _scmax(nbr_flat, bm2)

    midx = jnp.arange(MG, dtype=jnp.int32) * MMR
    f2 = batch[midx]
    l2 = batch[midx + MMR - 1]
    return _final(f2, l2, a2, g2, brow, b2r, wct, bcr)


# speedup vs baseline: 1.0770x; 1.0030x over previous
import functools

import jax
import jax.numpy as jnp
from jax import lax
from jax.experimental import pallas as pl
from jax.experimental.pallas import tpu as pltpu
import jax.experimental.pallas.tpu_sc as plsc

N = 16384
K = 16
NHID = 128
NCLS = 40
B = 16
NF = 3

RB = 128
CB = 384
G = N // RB
NCB = N // CB
MMR = 1024
MG = N // MMR
IBIG = 2 ** 30
HIGH = jax.lax.Precision.HIGHEST


def _knn_kernel(lo_ref, hi_ref, xtra_ref, pos_ref, post_ref, brow_ref,
                bcol_ref, out_ref, rv_ref, ri_ref):
    i = pl.program_id(0)
    qx = post_ref[0:1, :]
    qy = post_ref[1:2, :]
    qz = post_ref[2:3, :]
    bq = bcol_ref[...]

    rv_ref[...] = jnp.full((K, RB), jnp.inf, jnp.float32)
    ri_ref[...] = jnp.full((K, RB), IBIG, jnp.int32)

    def merge_block(cb):
        r0 = pl.multiple_of(cb * CB, CB)
        p = pos_ref[pl.ds(r0, CB), :]
        bs = brow_ref[pl.ds(r0, CB), :]
        tx = p[:, 0:1] - qx
        ty = p[:, 1:2] - qy
        tz = p[:, 2:3] - qz
        d = (tx * tx + ty * ty) + tz * tz
        d = jnp.where(bs == bq, d, jnp.inf)
        sidx = lax.broadcasted_iota(jnp.int32, (CB, RB), 0)
        rv = rv_ref[...]
        ri = ri_ref[...]
        ov = []
        oi = []
        for _ in range(K):
            m = jnp.minimum(jnp.min(d, axis=0, keepdims=True),
                            jnp.min(rv, axis=0, keepdims=True))
            cid = jnp.minimum(
                r0 + jnp.min(jnp.where(d == m, sidx, IBIG), axis=0,
                             keepdims=True),
                jnp.min(jnp.where(rv == m, ri, IBIG), axis=0, keepdims=True))
            ov.append(m)
            oi.append(cid)
            d = jnp.where(sidx == cid - r0, jnp.inf, d)
            rv = jnp.where(ri == cid, jnp.inf, rv)
        rv_ref[...] = jnp.concatenate(ov, axis=0)
        ri_ref[...] = jnp.concatenate(oi, axis=0)

    lo = lo_ref[i]
    hi = hi_ref[i]
    s0 = lo // CB
    s1 = (hi - 1) // CB

    def body(cb, carry):
        merge_block(cb)
        return carry

    lax.fori_loop(s0, s1 + 1, body, 0)

    @pl.when(xtra_ref[i] > 0)
    def _():
        merge_block(0)

    out_ref[...] = ri_ref[...]


def _knn(lo, hi, xtra, pos, post, brow, bcol):
    return pl.pallas_call(
        _knn_kernel,
        grid=(G,),
        in_specs=[
            pl.BlockSpec(memory_space=pltpu.SMEM),
            pl.BlockSpec(memory_space=pltpu.SMEM),
            pl.BlockSpec(memory_space=pltpu.SMEM),
            pl.BlockSpec((N, NF), lambda i: (0, 0)),
            pl.BlockSpec((NF, RB), lambda i: (0, i)),
            pl.BlockSpec((N, 1), lambda i: (0, 0)),
            pl.BlockSpec((1, RB), lambda i: (0, i)),
        ],
        out_specs=pl.BlockSpec((K, RB), lambda i: (0, i)),
        out_shape=jax.ShapeDtypeStruct((K, N), jnp.int32),
        scratch_shapes=[
            pltpu.VMEM((K, RB), jnp.float32),
            pltpu.VMEM((K, RB), jnp.int32),
        ],
    )(lo, hi, xtra, pos, post, brow, bcol)


def _mm1_kernel(pos_ref, ma_ref, mb_ref, a_ref, b_ref):
    q = pos_ref[...]

    def lin(m_ref):
        m = m_ref[...]
        return (q[:, 0:1] * m[0:1, :] + q[:, 1:2] * m[1:2, :]) + \
            q[:, 2:3] * m[2:3, :]

    a_ref[...] = lin(ma_ref)
    b_ref[...] = lin(mb_ref)


def _mm1(pos, m1a, m1b):
    return pl.pallas_call(
        _mm1_kernel,
        grid=(MG,),
        in_specs=[
            pl.BlockSpec((MMR, NF), lambda i: (i, 0)),
            pl.BlockSpec((NF, NHID), lambda i: (0, 0)),
            pl.BlockSpec((NF, NHID), lambda i: (0, 0)),
        ],
        out_specs=[
            pl.BlockSpec((MMR, NHID), lambda i: (i, 0)),
            pl.BlockSpec((MMR, NHID), lambda i: (i, 0)),
        ],
        out_shape=[
            jax.ShapeDtypeStruct((N, NHID), jnp.float32),
            jax.ShapeDtypeStruct((N, NHID), jnp.float32),
        ],
    )(pos, m1a, m1b)


def _mm2_kernel(a_ref, g_ref, b1_ref, ma_ref, mb_ref, a2_ref, b2_ref):
    h = jnp.maximum(a_ref[...] + g_ref[...] + b1_ref[...], 0.0)
    a2_ref[...] = jnp.dot(h, ma_ref[...],
                          preferred_element_type=jnp.float32, precision=HIGH)
    b2_ref[...] = jnp.dot(h, mb_ref[...],
                          preferred_element_type=jnp.float32, precision=HIGH)


def _mm2(a1, g1, b1, m2a, m2b):
    return pl.pallas_call(
        _mm2_kernel,
        grid=(MG,),
        in_specs=[
            pl.BlockSpec((MMR, NHID), lambda i: (i, 0)),
            pl.BlockSpec((MMR, NHID), lambda i: (i, 0)),
            pl.BlockSpec((1, NHID), lambda i: (0, 0)),
            pl.BlockSpec((NHID, NHID), lambda i: (0, 0)),
            pl.BlockSpec((NHID, NHID), lambda i: (0, 0)),
        ],
        out_specs=[
            pl.BlockSpec((MMR, NHID), lambda i: (i, 0)),
            pl.BlockSpec((MMR, NHID), lambda i: (i, 0)),
        ],
        out_shape=[
            jax.ShapeDtypeStruct((N, NHID), jnp.float32),
            jax.ShapeDtypeStruct((N, NHID), jnp.float32),
        ],
    )(a1, g1, b1, m2a, m2b)


def _final_kernel(first_ref, last_ref, a_ref, g_ref, brow_ref, b2_ref,
                  wc_ref, bc_ref, out_ref, pool_ref):
    i = pl.program_id(0)

    @pl.when(i == 0)
    def _():
        pool_ref[...] = jnp.full((B, NHID), -jnp.inf, jnp.float32)

    h = jnp.maximum(a_ref[...] + g_ref[...] + b2_ref[...], 0.0)
    br = brow_ref[...]

    def body(bb, carry):
        mx = jnp.max(jnp.where(br == bb, h, -jnp.inf), axis=0, keepdims=True)
        pool_ref[pl.ds(bb, 1), :] = jnp.maximum(pool_ref[pl.ds(bb, 1), :], mx)
        return carry

    lax.fori_loop(first_ref[i], last_ref[i] + 1, body, 0)

    @pl.when(i == pl.num_programs(0) - 1)
    def _():
        out_ref[...] = jnp.dot(pool_ref[...], wc_ref[...],
                               preferred_element_type=jnp.float32,
                               precision=HIGH) + bc_ref[...]


def _final(first, last, a2, g2, brow, b2, wct, bc):
    return pl.pallas_call(
        _final_kernel,
        grid=(MG,),
        in_specs=[
            pl.BlockSpec(memory_space=pltpu.SMEM),
            pl.BlockSpec(memory_space=pltpu.SMEM),
            pl.BlockSpec((MMR, NHID), lambda i: (i, 0)),
            pl.BlockSpec((MMR, NHID), lambda i: (i, 0)),
            pl.BlockSpec((MMR, 1), lambda i: (i, 0)),
            pl.BlockSpec((1, NHID), lambda i: (0, 0)),
            pl.BlockSpec((NHID, NCLS), lambda i: (0, 0)),
            pl.BlockSpec((1, NCLS), lambda i: (0, 0)),
        ],
        out_specs=pl.BlockSpec((B, NCLS), lambda i: (0, 0)),
        out_shape=jax.ShapeDtypeStruct((B, NCLS), jnp.float32),
        scratch_shapes=[pltpu.VMEM((B, NHID), jnp.float32)],
    )(first, last, a2, g2, brow, b2, wct, bc)


_NC = 2
_NS = 16
_NW = _NC * _NS
_RPW = N // _NW
_CH = 8
_NCHUNK = _RPW // _CH

_NBUF = 2


def _scmax_body(nbr_ref, bm_ref, out_ref, idx_v, rows0, rows1,
                ob0, ob1, gsem0, gsem1, osem0, osem1):
    wid = lax.axis_index("s") * _NC + lax.axis_index("c")
    base = wid * _RPW
    rows = (rows0, rows1)
    obs = (ob0, ob1)
    gsems = (gsem0, gsem1)
    osems = (osem0, osem1)

    pltpu.sync_copy(nbr_ref.at[pl.ds(base * K, _RPW * K)], idx_v)

    def gstart(ch, buf, sem):
        pltpu.async_copy(bm_ref.at[idx_v.at[pl.ds(ch * _CH * K, _CH * K)]],
                         buf, sem)

    def gwait(buf, sem):
        pltpu.make_async_copy(bm_ref.at[idx_v.at[pl.ds(0, _CH * K)]],
                              buf, sem).wait()

    def owait(ob, sem):
        pltpu.make_async_copy(ob, out_ref.at[pl.ds(base, _CH)], sem).wait()

    def compute(rows_v, out_v):
        for dl in range(_CH):
            for g in range(NHID // 16):
                acc = rows_v[dl * K, pl.ds(g * 16, 16)]
                for j in range(1, K):
                    acc = jnp.maximum(acc,
                                      rows_v[dl * K + j, pl.ds(g * 16, 16)])
                out_v[dl, pl.ds(g * 16, 16)] = acc

    for b in range(_NBUF):
        gstart(b, rows[b], gsems[b])

    def grp(p, carry):
        ch0 = _NBUF * p
        for b in range(_NBUF):
            ch = ch0 + b
            gwait(rows[b], gsems[b])

            @pl.when(p > 0)
            def _():
                owait(obs[b], osems[b])

            compute(rows[b], obs[b])
            pltpu.async_copy(obs[b], out_ref.at[pl.ds(base + ch * _CH, _CH)],
                             osems[b])

            @pl.when(p < _NCHUNK // _NBUF - 1)
            def _():
                gstart(ch + _NBUF, rows[b], gsems[b])
        return carry

    lax.fori_loop(0, _NCHUNK // _NBUF, grp, 0)
    for b in range(_NBUF):
        owait(obs[b], osems[b])


@functools.lru_cache(maxsize=1)
def _scmax_fn():
    mesh = plsc.VectorSubcoreMesh(
        core_axis_name="c", subcore_axis_name="s",
        num_cores=_NC, num_subcores=_NS)
    return pl.kernel(
        _scmax_body,
        out_type=jax.ShapeDtypeStruct((N, NHID), jnp.float32),
        mesh=mesh,
        scratch_types=(
            [pltpu.VMEM((_RPW * K,), jnp.int32)]
            + [pltpu.VMEM((_CH * K, NHID), jnp.float32)] * _NBUF
            + [pltpu.VMEM((_CH, NHID), jnp.float32)] * _NBUF
            + [pltpu.SemaphoreType.DMA] * (2 * _NBUF)
        ),
    )


def _scmax(nbr_flat, bm):
    return _scmax_fn()(nbr_flat, bm)


def kernel(pos, batch, W1, b1, W2, b2, Wc, bc):
    pos = pos.astype(jnp.float32)
    batch = batch.astype(jnp.int32)

    post = pos.T
    brow = batch.reshape(N, 1)
    bcol = batch.reshape(1, N)

    bnd = jnp.searchsorted(
        batch, jnp.arange(B + 1, dtype=jnp.int32), side="left"
    ).astype(jnp.int32)
    ridx = jnp.arange(G, dtype=jnp.int32) * RB
    first = batch[ridx]
    last = batch[ridx + RB - 1]
    lo = bnd[first]
    hi = bnd[last + 1]
    seg = bnd[1:] - bnd[:-1]
    bb = jnp.arange(B, dtype=jnp.int32)
    inblk = (bb[None, :] >= first[:, None]) & (bb[None, :] <= last[:, None])
    minseg = jnp.min(jnp.where(inblk, seg[None, :], IBIG), axis=1)
    xtra = (minseg < K).astype(jnp.int32)

    m1a = (W1[:, :NF] - W1[:, NF:]).T
    m1b = W1[:, NF:].T
    m2a = (W2[:, :NHID] - W2[:, NHID:]).T
    m2b = W2[:, NHID:].T
    wct = Wc.T
    b1r = b1.reshape(1, NHID)
    b2r = b2.reshape(1, NHID)
    bcr = bc.reshape(1, NCLS)

    nbr = _knn(lo, hi, xtra, pos, post, brow, bcol)
    nbr_flat = nbr.T.reshape(-1)

    a1, bm1 = _mm1(pos, m1a, m1b)
    g1 = _scmax(nbr_flat, bm1)
    a2, bm2 = _mm2(a1, g1, b1r, m2a, m2b)
    g2 = _scmax(nbr_flat, bm2)

    midx = jnp.arange(MG, dtype=jnp.int32) * MMR
    f2 = batch[midx]
    l2 = batch[midx + MMR - 1]
    return _final(f2, l2, a2, g2, brow, b2r, wct, bcr)
